# baseline (speedup 1.0000x reference)
import functools

import jax
import jax.numpy as jnp
from jax import lax
from jax.experimental import pallas as pl
from jax.experimental.pallas import tpu as pltpu
from jax.experimental.pallas import tpu_sc as plsc

B = 4096
F = 26
D = 32
NFLAT = B * F
NC, NS = 2, 16
NW = NC * NS
PER_W = NFLAT // NW

BN_EPS = 1e-5

_mesh = plsc.VectorSubcoreMesh(core_axis_name="c", subcore_axis_name="s")


@functools.partial(
    pl.kernel,
    mesh=_mesh,
    out_type=(
        jax.ShapeDtypeStruct((NFLAT, D), jnp.float32),
        jax.ShapeDtypeStruct((NFLAT, 1), jnp.float32),
    ),
    scratch_types=[
        pltpu.VMEM((PER_W // 2,), jnp.int32),
        pltpu.VMEM((PER_W // 2, D), jnp.float32),
        pltpu.VMEM((PER_W // 2, 1), jnp.float32),
        pltpu.SemaphoreType.DMA,
        pltpu.SemaphoreType.DMA,
    ],
    compiler_params=pltpu.CompilerParams(use_tc_tiling_on_sc=False),
)
def _sc_gather(idx_hbm, emb_tab, o1_tab, emb_out, o1_out,
               idx_v, rows_v, o1_v, sem1, sem2):
    wid = lax.axis_index("s") * NC + lax.axis_index("c")
    ch = PER_W // 2
    for c in range(2):
        base = wid * PER_W + c * ch
        pltpu.sync_copy(idx_hbm.at[pl.ds(base, ch)], idx_v)
        c1 = pltpu.async_copy(emb_tab.at[idx_v], rows_v, sem1)
        c2 = pltpu.async_copy(o1_tab.at[idx_v], o1_v, sem2)
        c1.wait()
        pltpu.sync_copy(rows_v, emb_out.at[pl.ds(base, ch)])
        c2.wait()
        pltpu.sync_copy(o1_v, o1_out.at[pl.ds(base, ch)])


def _tc_body(emb_ref, o1v_ref, W1_ref, b1_ref, g1_ref, bt1_ref,
             W2_ref, b2_ref, g2_ref, bt2_ref, W3_ref, b3_ref,
             W4_ref, b4_ref, out_ref):
    emb = emb_ref[...]
    s = emb[:, 0:D]
    for f in range(1, F):
        s = s + emb[:, f * D:(f + 1) * D]
    sq_of_sum = jnp.sum(s * s, axis=1, keepdims=True)
    sum_of_sq = jnp.sum(emb * emb, axis=1, keepdims=True)
    o2 = 0.5 * (sq_of_sum - sum_of_sq)
    o1 = jnp.sum(o1v_ref[...], axis=1, keepdims=True)
    h = jnp.dot(emb, W1_ref[...], preferred_element_type=jnp.float32) + b1_ref[...]
    mu = jnp.mean(h, axis=0, keepdims=True)
    var = jnp.mean((h - mu) ** 2, axis=0, keepdims=True)
    h = (h - mu) / jnp.sqrt(var + BN_EPS) * g1_ref[...] + bt1_ref[...]
    h = jnp.maximum(h, 0.0)
    h = jnp.dot(h, W2_ref[...], preferred_element_type=jnp.float32) + b2_ref[...]
    mu = jnp.mean(h, axis=0, keepdims=True)
    var = jnp.mean((h - mu) ** 2, axis=0, keepdims=True)
    h = (h - mu) / jnp.sqrt(var + BN_EPS) * g2_ref[...] + bt2_ref[...]
    h = jnp.maximum(h, 0.0)
    h = jnp.dot(h, W3_ref[...], preferred_element_type=jnp.float32) + b3_ref[...]
    dnn = jnp.dot(h, W4_ref[...], preferred_element_type=jnp.float32) + b4_ref[...]
    out_ref[...] = o1 + o2 + dnn


def kernel(x, cat_embed, o1_table, W1, b1, g1, bt1, W2, b2, g2, bt2,
           W3, b3, W4, b4):
    idx = x.reshape(-1).astype(jnp.int32)
    emb_flat, o1_flat = _sc_gather(idx, cat_embed, o1_table)
    emb2d = emb_flat.reshape(B, F * D)
    o1v = o1_flat.reshape(B, F)
    out = pl.pallas_call(
        _tc_body,
        out_shape=jax.ShapeDtypeStruct((B, 1), jnp.float32),
    )(emb2d, o1v, W1, b1.reshape(1, -1), g1.reshape(1, -1), bt1.reshape(1, -1),
      W2, b2.reshape(1, -1), g2.reshape(1, -1), bt2.reshape(1, -1),
      W3, b3.reshape(1, -1), W4, b4.reshape(1, -1))
    return out


# speedup vs baseline: 14.1763x; 14.1763x over previous
import functools

import jax
import jax.numpy as jnp
from jax import lax
from jax.experimental import pallas as pl
from jax.experimental.pallas import tpu as pltpu
from jax.experimental.pallas import tpu_sc as plsc

B = 4096
F = 26
D = 32
NFLAT = B * F
NC, NS = 2, 16
NW = NC * NS
PER_W = NFLAT // NW

BN_EPS = 1e-5

_mesh = plsc.VectorSubcoreMesh(core_axis_name="c", subcore_axis_name="s")


@functools.partial(
    pl.kernel,
    mesh=_mesh,
    out_type=jax.ShapeDtypeStruct((NFLAT,), jnp.float32),
    scratch_types=[
        pltpu.VMEM((PER_W,), jnp.int32),
        pltpu.VMEM((PER_W,), jnp.float32),
        pltpu.SemaphoreType.DMA,
    ],
    compiler_params=pltpu.CompilerParams(use_tc_tiling_on_sc=False),
)
def _sc_o1_gather(idx_hbm, o1_tab, o1_out, idx_v, o1_v, sem):
    wid = lax.axis_index("s") * NC + lax.axis_index("c")
    base = wid * PER_W
    pltpu.sync_copy(idx_hbm.at[pl.ds(base, PER_W)], idx_v)
    pltpu.async_copy(o1_tab.at[idx_v], o1_v, sem).wait()
    pltpu.sync_copy(o1_v, o1_out.at[pl.ds(base, PER_W)])


def _tc_body(emb_ref, o1v_ref, W1_ref, b1_ref, g1_ref, bt1_ref,
             W2_ref, b2_ref, g2_ref, bt2_ref, W3_ref, b3_ref,
             W4_ref, b4_ref, out_ref):
    emb = emb_ref[...]
    s = emb[:, 0:D]
    for f in range(1, F):
        s = s + emb[:, f * D:(f + 1) * D]
    sq_of_sum = jnp.sum(s * s, axis=1, keepdims=True)
    sum_of_sq = jnp.sum(emb * emb, axis=1, keepdims=True)
    o2 = 0.5 * (sq_of_sum - sum_of_sq)
    o1 = jnp.sum(o1v_ref[...], axis=1, keepdims=True)
    h = jnp.dot(emb, W1_ref[...], preferred_element_type=jnp.float32) + b1_ref[...]
    mu = jnp.mean(h, axis=0, keepdims=True)
    var = jnp.mean((h - mu) ** 2, axis=0, keepdims=True)
    h = (h - mu) / jnp.sqrt(var + BN_EPS) * g1_ref[...] + bt1_ref[...]
    h = jnp.maximum(h, 0.0)
    h = jnp.dot(h, W2_ref[...], preferred_element_type=jnp.float32) + b2_ref[...]
    mu = jnp.mean(h, axis=0, keepdims=True)
    var = jnp.mean((h - mu) ** 2, axis=0, keepdims=True)
    h = (h - mu) / jnp.sqrt(var + BN_EPS) * g2_ref[...] + bt2_ref[...]
    h = jnp.maximum(h, 0.0)
    h = jnp.dot(h, W3_ref[...], preferred_element_type=jnp.float32) + b3_ref[...]
    dnn = jnp.dot(h, W4_ref[...], preferred_element_type=jnp.float32) + b4_ref[...]
    out_ref[...] = o1 + o2 + dnn


def kernel(x, cat_embed, o1_table, W1, b1, g1, bt1, W2, b2, g2, bt2,
           W3, b3, W4, b4):
    idx = x.reshape(-1).astype(jnp.int32)
    o1_flat = _sc_o1_gather(idx, o1_table[:, 0])
    emb2d = jnp.take(cat_embed, idx, axis=0).reshape(B, F * D)
    o1v = o1_flat.reshape(B, F)
    out = pl.pallas_call(
        _tc_body,
        out_shape=jax.ShapeDtypeStruct((B, 1), jnp.float32),
    )(emb2d, o1v, W1, b1.reshape(1, -1), g1.reshape(1, -1), bt1.reshape(1, -1),
      W2, b2.reshape(1, -1), g2.reshape(1, -1), bt2.reshape(1, -1),
      W3, b3.reshape(1, -1), W4, b4.reshape(1, -1))
    return out
